# baseline (speedup 1.0000x reference)
import functools

import jax
import jax.numpy as jnp
from jax import lax
from jax.experimental import pallas as pl
from jax.experimental.pallas import tpu as pltpu
from jax.experimental.pallas import tpu_sc as plsc

N_DST0 = 20000
N_DST1 = 5000


def _elu(x):
    return jnp.where(x > 0, x, jnp.exp(x) - 1.0)


def _mlp_body(nlayers, has_scale, final_act, split_out, *refs):
    x_ref = refs[0]
    nin = 1 + 2 * nlayers + (1 if has_scale else 0)
    scale_ref = refs[nin - 1] if has_scale else None
    x = x_ref[...]
    for i in range(nlayers):
        w = refs[1 + 2 * i][...]
        b = refs[2 + 2 * i][...]
        x = jnp.dot(x, w, preferred_element_type=jnp.float32) + b
        if i < nlayers - 1 or final_act:
            x = _elu(x)
    if has_scale:
        x = x * scale_ref[...]
    if split_out:
        half = x.shape[1] // 2
        refs[nin][...] = x[:, :half]
        refs[nin + 1][...] = x[:, half:]
    else:
        refs[nin][...] = x


def _pick_block(n):
    for b in (2000, 1000, 500, 250, 200, 100, 50, 25, 20, 10, 8, 5, 4, 2, 1):
        if n % b == 0:
            return b
    return 1


def _run_mlp(x, layers, *, final_act=False, scale=None, split_out=False):
    n = x.shape[0]
    bn = _pick_block(n)
    grid = (n // bn,)
    nlayers = len(layers)
    odim = layers[-1]["w"].shape[1]

    in_specs = [pl.BlockSpec((bn, x.shape[1]), lambda i: (i, 0))]
    args = [x]
    for p in layers:
        w = p["w"]
        b = p["b"].reshape(1, -1)
        in_specs.append(pl.BlockSpec(w.shape, lambda i: (0, 0)))
        in_specs.append(pl.BlockSpec(b.shape, lambda i: (0, 0)))
        args.append(w)
        args.append(b)
    if scale is not None:
        in_specs.append(pl.BlockSpec((bn, 1), lambda i: (i, 0)))
        args.append(scale)

    if split_out:
        half = odim // 2
        out_specs = [pl.BlockSpec((bn, half), lambda i: (i, 0)),
                     pl.BlockSpec((bn, half), lambda i: (i, 0))]
        out_shape = [jax.ShapeDtypeStruct((n, half), jnp.float32),
                     jax.ShapeDtypeStruct((n, half), jnp.float32)]
    else:
        out_specs = pl.BlockSpec((bn, odim), lambda i: (i, 0))
        out_shape = jax.ShapeDtypeStruct((n, odim), jnp.float32)

    out = pl.pallas_call(
        functools.partial(_mlp_body, nlayers, scale is not None, final_act, split_out),
        grid=grid,
        in_specs=in_specs,
        out_specs=out_specs,
        out_shape=out_shape,
    )(*args)
    return out


def _sc_agg(feat_a, feat_b, e_a, e_b, src2d, dst2d, zeros, n_dst, G, NI):
    nsg_total = src2d.shape[0] // NI
    nsg = nsg_total // 16
    src3d = src2d.reshape(nsg_total, NI, src2d.shape[1])
    dst3d = dst2d.reshape(nsg_total, NI, dst2d.shape[1])
    e_a3 = e_a.reshape(nsg_total, NI * G, 64)
    e_b3 = e_b.reshape(nsg_total, NI * G, 64)
    zc = 2000 if n_dst % 2000 == 0 else 1000
    nz = n_dst // zc

    assert nsg % 2 == 0 and nsg >= 4

    def body(fa, fb, ea, eb, srch, dsth, zh, out,
             src_v0, dst_v0, rows_v0, e_v0,
             src_v1, dst_v1, rows_v1, e_v1,
             acc, gsem0, gsem1, ssem0, ssem1):
        c = lax.axis_index("c")
        s = lax.axis_index("s")
        bufs = ((src_v0, dst_v0, rows_v0, e_v0, gsem0, ssem0),
                (src_v1, dst_v1, rows_v1, e_v1, gsem1, ssem1))

        @pl.when(s < nz)
        def _zero():
            pltpu.sync_copy(zh.at[pl.ds(s * zc, zc)], acc.at[pl.ds(s * zc, zc)])
        plsc.subcore_barrier()

        def load_and_fire(sgi, p):
            src_v, dst_v, rows_v, e_v, gsem, _ = bufs[p]
            pltpu.sync_copy(srch.at[sgi], src_v)
            pltpu.sync_copy(dsth.at[sgi], dst_v)

            @pl.when(c == 0)
            def _():
                pltpu.sync_copy(ea.at[sgi], e_v)
                for j in range(NI):
                    pltpu.async_copy(fa.at[src_v.at[j]],
                                     rows_v.at[pl.ds(j * G, G)], gsem)

            @pl.when(c == 1)
            def _():
                pltpu.sync_copy(eb.at[sgi], e_v)
                for j in range(NI):
                    pltpu.async_copy(fb.at[src_v.at[j]],
                                     rows_v.at[pl.ds(j * G, G)], gsem)

        def drain_gathers(p):
            _, _, rows_v, _, gsem, _ = bufs[p]
            for j in range(NI):
                pltpu.make_async_copy(fa.at[pl.ds(0, G)],
                                      rows_v.at[pl.ds(j * G, G)], gsem).wait()

        def drain_scatters(p):
            _, _, rows_v, _, _, ssem = bufs[p]
            for j in range(NI):
                pltpu.make_async_copy(fa.at[pl.ds(0, G)],
                                      rows_v.at[pl.ds(j * G, G)], ssem).wait()

        def compute_and_scatter(p):
            _, dst_v, rows_v, e_v, _, ssem = bufs[p]

            def mul(g, carry2):
                g4 = g * 4
                for gg in range(4):
                    for k4 in range(4):
                        sl = pl.ds(k4 * 16, 16)
                        rows_v[g4 + gg, sl] = rows_v[g4 + gg, sl] * e_v[g4 + gg, sl]
                return carry2
            lax.fori_loop(0, (NI * G) // 4, mul, 0)
            for j in range(NI):
                pltpu.async_copy(rows_v.at[pl.ds(j * G, G)],
                                 acc.at[dst_v.at[j]], ssem, add=True)

        sg0 = s * nsg
        load_and_fire(sg0, 0)

        def step(k, carry):
            @pl.when(k > 0)
            def _():
                drain_scatters(1)
            load_and_fire(sg0 + 2 * k + 1, 1)
            drain_gathers(0)
            compute_and_scatter(0)
            drain_scatters(0)

            @pl.when(2 * k + 2 < nsg)
            def _():
                load_and_fire(sg0 + 2 * k + 2, 0)
            drain_gathers(1)
            compute_and_scatter(1)
            return carry

        lax.fori_loop(0, nsg // 2, step, 0)
        drain_scatters(1)

        plsc.subcore_barrier()

        @pl.when(s < nz)
        def _writeout():
            pltpu.sync_copy(acc.at[pl.ds(s * zc, zc)],
                            out.at[c, pl.ds(s * zc, zc)])

    out = pl.kernel(
        body,
        out_type=jax.ShapeDtypeStruct((2, n_dst, 64), jnp.float32),
        mesh=plsc.VectorSubcoreMesh(core_axis_name="c", subcore_axis_name="s"),
        compiler_params=pltpu.CompilerParams(use_tc_tiling_on_sc=False),
        scratch_types=[
            pltpu.VMEM((NI, G), jnp.int32),
            pltpu.VMEM((NI, G), jnp.int32),
            pltpu.VMEM((NI * G, 64), jnp.float32),
            pltpu.VMEM((NI * G, 64), jnp.float32),
            pltpu.VMEM((NI, G), jnp.int32),
            pltpu.VMEM((NI, G), jnp.int32),
            pltpu.VMEM((NI * G, 64), jnp.float32),
            pltpu.VMEM((NI * G, 64), jnp.float32),
            pltpu.VMEM_SHARED((n_dst, 64), jnp.float32),
            pltpu.SemaphoreType.DMA,
            pltpu.SemaphoreType.DMA,
            pltpu.SemaphoreType.DMA,
            pltpu.SemaphoreType.DMA,
        ],
    )(feat_a, feat_b, e_a3, e_b3, src3d, dst3d, zeros)
    return jnp.concatenate([out[0], out[1]], axis=1)


def _softmax_w(dst, offsets, num_dst):
    dist = 1.0 / (jnp.sum(jnp.abs(offsets), axis=1) + 0.001)
    mx = jax.ops.segment_max(dist, dst, num_segments=num_dst)
    ex = jnp.exp(dist - mx[dst])
    denom = jax.ops.segment_sum(ex, dst, num_segments=num_dst)
    return ex / denom[dst]


def _coord_conv(p, feat_ab, feat_dst, src2d, dst2d, offsets, w, num_dst, zeros, G, NI):
    e_a, e_b = _run_mlp(offsets, p["kernel"], scale=w[:, None], split_out=True)
    agg = _sc_agg(feat_ab[0], feat_ab[1], e_a, e_b, src2d, dst2d, zeros, num_dst, G, NI)
    x_self = _run_mlp(feat_dst, p["mlp_self"])
    return _run_mlp(jnp.concatenate([agg, x_self], axis=1), p["mlp"])


def kernel(feat, src0, dst0, offsets0, src1, dst1, offsets1, num_dst0, num_dst1, params):
    feat = feat + (jnp.asarray(num_dst0) - N_DST0).astype(jnp.float32) \
                + (jnp.asarray(num_dst1) - N_DST1).astype(jnp.float32)

    def sk(name, x):
        return _run_mlp(x, [params[name]])

    fd0 = feat[:N_DST0]
    h0 = sk("skip1", fd0); h0_ = sk("skip2", fd0); h0__ = sk("skip3", fd0)
    fd1 = feat[:N_DST1]
    h2 = sk("skip4", fd1); h2_ = sk("skip5", fd1); h2__ = sk("skip6", fd1)

    G0, NI0 = 40, 2
    G1, NI1 = 100, 1
    src0_2d = src0.reshape(-1, G0); dst0_2d = dst0.reshape(-1, G0)
    src1_2d = src1.reshape(-1, G1); dst1_2d = dst1.reshape(-1, G1)
    z0 = jnp.zeros((N_DST0, 64), jnp.float32)
    z1 = jnp.zeros((N_DST1, 64), jnp.float32)
    f_ab0 = (feat[:, :64], feat[:, 64:])

    w_lvl0 = _softmax_w(dst0, offsets0, N_DST0)
    h = _coord_conv(params["conv1"], f_ab0, fd0, src0_2d, dst0_2d, offsets0, w_lvl0, N_DST0, z0, G0, NI0) + h0
    h_ = _coord_conv(params["conv2"], f_ab0, fd0, src0_2d, dst0_2d, offsets0, w_lvl0, N_DST0, z0, G0, NI0) + h0_
    h__ = _coord_conv(params["conv3"], f_ab0, fd0, src0_2d, dst0_2d, offsets0, w_lvl0, N_DST0, z0, G0, NI0) + h0__

    h1 = h[:N_DST1]; h1_ = h_[:N_DST1]; h1__ = h__[:N_DST1]
    h = _elu(h); h_ = _elu(h_); h__ = _elu(h__)

    w_lvl1 = _softmax_w(dst1, offsets1, N_DST1)
    h = _coord_conv(params["conv4"], (h[:, :64], h[:, 64:]), h[:N_DST1], src1_2d, dst1_2d, offsets1, w_lvl1, N_DST1, z1, G1, NI1) + h1
    h_ = _coord_conv(params["conv5"], (h_[:, :64], h_[:, 64:]), h_[:N_DST1], src1_2d, dst1_2d, offsets1, w_lvl1, N_DST1, z1, G1, NI1) + h1_
    h__ = _coord_conv(params["conv6"], (h__[:, :64], h__[:, 64:]), h__[:N_DST1], src1_2d, dst1_2d, offsets1, w_lvl1, N_DST1, z1, G1, NI1) + h1__

    h = _elu(jnp.concatenate([h, h2], axis=1))
    h_ = _elu(jnp.concatenate([h_, h2_], axis=1))
    h__ = _elu(jnp.concatenate([h__, h2__], axis=1))

    out1 = _run_mlp(h, params["out1"])
    out2 = _run_mlp(h_, params["out2"])
    out3 = _run_mlp(h__, params["out3"])
    return jnp.concatenate([out1, out2, out3], axis=1)


# speedup vs baseline: 1.0270x; 1.0270x over previous
import functools

import jax
import jax.numpy as jnp
from jax import lax
from jax.experimental import pallas as pl
from jax.experimental.pallas import tpu as pltpu
from jax.experimental.pallas import tpu_sc as plsc

N_DST0 = 20000
N_DST1 = 5000


def _elu(x):
    return jnp.where(x > 0, x, jnp.exp(x) - 1.0)


def _mlp_body(nlayers, has_scale, final_act, split_out, *refs):
    x_ref = refs[0]
    nin = 1 + 2 * nlayers + (1 if has_scale else 0)
    scale_ref = refs[nin - 1] if has_scale else None
    x = x_ref[...]
    for i in range(nlayers):
        w = refs[1 + 2 * i][...]
        b = refs[2 + 2 * i][...]
        x = jnp.dot(x, w, preferred_element_type=jnp.float32) + b
        if i < nlayers - 1 or final_act:
            x = _elu(x)
    if has_scale:
        x = x * scale_ref[...]
    if split_out:
        half = x.shape[1] // 2
        refs[nin][...] = x[:, :half]
        refs[nin + 1][...] = x[:, half:]
    else:
        refs[nin][...] = x


def _pick_block(n):
    for b in (2000, 1000, 500, 250, 200, 100, 50, 25, 20, 10, 8, 5, 4, 2, 1):
        if n % b == 0:
            return b
    return 1


def _run_mlp(x, layers, *, final_act=False, scale=None, split_out=False):
    n = x.shape[0]
    bn = _pick_block(n)
    grid = (n // bn,)
    nlayers = len(layers)
    odim = layers[-1]["w"].shape[1]

    in_specs = [pl.BlockSpec((bn, x.shape[1]), lambda i: (i, 0))]
    args = [x]
    for p in layers:
        w = p["w"]
        b = p["b"].reshape(1, -1)
        in_specs.append(pl.BlockSpec(w.shape, lambda i: (0, 0)))
        in_specs.append(pl.BlockSpec(b.shape, lambda i: (0, 0)))
        args.append(w)
        args.append(b)
    if scale is not None:
        in_specs.append(pl.BlockSpec((bn, 1), lambda i: (i, 0)))
        args.append(scale)

    if split_out:
        half = odim // 2
        out_specs = [pl.BlockSpec((bn, half), lambda i: (i, 0)),
                     pl.BlockSpec((bn, half), lambda i: (i, 0))]
        out_shape = [jax.ShapeDtypeStruct((n, half), jnp.float32),
                     jax.ShapeDtypeStruct((n, half), jnp.float32)]
    else:
        out_specs = pl.BlockSpec((bn, odim), lambda i: (i, 0))
        out_shape = jax.ShapeDtypeStruct((n, odim), jnp.float32)

    out = pl.pallas_call(
        functools.partial(_mlp_body, nlayers, scale is not None, final_act, split_out),
        grid=grid,
        in_specs=in_specs,
        out_specs=out_specs,
        out_shape=out_shape,
    )(*args)
    return out


def _sc_agg(feat_a, feat_b, e_a, e_b, src2d, dst2d, zeros, n_dst, G, NI):
    nsg_total = src2d.shape[0] // NI
    nsg = nsg_total // 16
    src3d = src2d.reshape(nsg_total, NI, src2d.shape[1])
    dst3d = dst2d.reshape(nsg_total, NI, dst2d.shape[1])
    e_a3 = e_a.reshape(nsg_total, NI * G, 64)
    e_b3 = e_b.reshape(nsg_total, NI * G, 64)
    zc = 2000 if n_dst % 2000 == 0 else 1000
    nz = n_dst // zc

    assert nsg % 2 == 0 and nsg >= 4

    def body(fa, fb, ea, eb, srch, dsth, zh, out,
             src_v0, dst_v0, rows_v0, e_v0,
             src_v1, dst_v1, rows_v1, e_v1,
             acc, gsem0, gsem1, ssem0, ssem1):
        c = lax.axis_index("c")
        s = lax.axis_index("s")
        bufs = ((src_v0, dst_v0, rows_v0, e_v0, gsem0, ssem0),
                (src_v1, dst_v1, rows_v1, e_v1, gsem1, ssem1))

        @pl.when(s < nz)
        def _zero():
            pltpu.sync_copy(zh.at[pl.ds(s * zc, zc)], acc.at[pl.ds(s * zc, zc)])
        plsc.subcore_barrier()

        def load_and_fire(sgi, p):
            src_v, dst_v, rows_v, e_v, gsem, _ = bufs[p]
            pltpu.sync_copy(srch.at[sgi], src_v)
            pltpu.sync_copy(dsth.at[sgi], dst_v)

            @pl.when(c == 0)
            def _():
                pltpu.sync_copy(ea.at[sgi], e_v)
                for j in range(NI):
                    pltpu.async_copy(fa.at[src_v.at[j]],
                                     rows_v.at[pl.ds(j * G, G)], gsem)

            @pl.when(c == 1)
            def _():
                pltpu.sync_copy(eb.at[sgi], e_v)
                for j in range(NI):
                    pltpu.async_copy(fb.at[src_v.at[j]],
                                     rows_v.at[pl.ds(j * G, G)], gsem)

        def drain_gathers(p):
            _, _, rows_v, _, gsem, _ = bufs[p]
            for j in range(NI):
                pltpu.make_async_copy(fa.at[pl.ds(0, G)],
                                      rows_v.at[pl.ds(j * G, G)], gsem).wait()

        def drain_scatters(p):
            _, _, rows_v, _, _, ssem = bufs[p]
            for j in range(NI):
                pltpu.make_async_copy(fa.at[pl.ds(0, G)],
                                      rows_v.at[pl.ds(j * G, G)], ssem).wait()

        def compute_and_scatter(p):
            _, dst_v, rows_v, e_v, _, ssem = bufs[p]

            def mul(g, carry2):
                g4 = g * 4
                for gg in range(4):
                    for k4 in range(4):
                        sl = pl.ds(k4 * 16, 16)
                        rows_v[g4 + gg, sl] = rows_v[g4 + gg, sl] * e_v[g4 + gg, sl]
                return carry2
            lax.fori_loop(0, (NI * G) // 4, mul, 0)
            for j in range(NI):
                pltpu.async_copy(rows_v.at[pl.ds(j * G, G)],
                                 acc.at[dst_v.at[j]], ssem, add=True)

        sg0 = s * nsg
        load_and_fire(sg0, 0)

        def step(k, carry):
            @pl.when(k > 0)
            def _():
                drain_scatters(1)
            load_and_fire(sg0 + 2 * k + 1, 1)
            drain_gathers(0)
            compute_and_scatter(0)
            drain_scatters(0)

            @pl.when(2 * k + 2 < nsg)
            def _():
                load_and_fire(sg0 + 2 * k + 2, 0)
            drain_gathers(1)
            compute_and_scatter(1)
            return carry

        lax.fori_loop(0, nsg // 2, step, 0)
        drain_scatters(1)

        plsc.subcore_barrier()

        @pl.when(s < nz)
        def _writeout():
            pltpu.sync_copy(acc.at[pl.ds(s * zc, zc)],
                            out.at[c, pl.ds(s * zc, zc)])

    out = pl.kernel(
        body,
        out_type=jax.ShapeDtypeStruct((2, n_dst, 64), jnp.float32),
        mesh=plsc.VectorSubcoreMesh(core_axis_name="c", subcore_axis_name="s"),
        compiler_params=pltpu.CompilerParams(use_tc_tiling_on_sc=False),
        scratch_types=[
            pltpu.VMEM((NI, G), jnp.int32),
            pltpu.VMEM((NI, G), jnp.int32),
            pltpu.VMEM((NI * G, 64), jnp.float32),
            pltpu.VMEM((NI * G, 64), jnp.float32),
            pltpu.VMEM((NI, G), jnp.int32),
            pltpu.VMEM((NI, G), jnp.int32),
            pltpu.VMEM((NI * G, 64), jnp.float32),
            pltpu.VMEM((NI * G, 64), jnp.float32),
            pltpu.VMEM_SHARED((n_dst, 64), jnp.float32),
            pltpu.SemaphoreType.DMA,
            pltpu.SemaphoreType.DMA,
            pltpu.SemaphoreType.DMA,
            pltpu.SemaphoreType.DMA,
        ],
    )(feat_a, feat_b, e_a3, e_b3, src3d, dst3d, zeros)
    return jnp.concatenate([out[0], out[1]], axis=1)


def _softmax_w(dst, offsets, num_dst):
    dist = 1.0 / (jnp.sum(jnp.abs(offsets), axis=1) + 0.001)
    mx = jax.ops.segment_max(dist, dst, num_segments=num_dst)
    ex = jnp.exp(dist - mx[dst])
    denom = jax.ops.segment_sum(ex, dst, num_segments=num_dst)
    return ex / denom[dst]


def _coord_conv(p, feat_ab, feat_dst, src2d, dst2d, offsets, w, num_dst, zeros, G, NI):
    e_a, e_b = _run_mlp(offsets, p["kernel"], scale=w[:, None], split_out=True)
    agg = _sc_agg(feat_ab[0], feat_ab[1], e_a, e_b, src2d, dst2d, zeros, num_dst, G, NI)
    x_self = _run_mlp(feat_dst, p["mlp_self"])
    return _run_mlp(jnp.concatenate([agg, x_self], axis=1), p["mlp"])


def kernel(feat, src0, dst0, offsets0, src1, dst1, offsets1, num_dst0, num_dst1, params):
    feat = feat + (jnp.asarray(num_dst0) - N_DST0).astype(jnp.float32) \
                + (jnp.asarray(num_dst1) - N_DST1).astype(jnp.float32)

    def sk(name, x):
        return _run_mlp(x, [params[name]])

    fd0 = feat[:N_DST0]
    h0 = sk("skip1", fd0); h0_ = sk("skip2", fd0); h0__ = sk("skip3", fd0)
    fd1 = feat[:N_DST1]
    h2 = sk("skip4", fd1); h2_ = sk("skip5", fd1); h2__ = sk("skip6", fd1)

    G0, NI0 = 100, 1
    G1, NI1 = 100, 1
    src0_2d = src0.reshape(-1, G0); dst0_2d = dst0.reshape(-1, G0)
    src1_2d = src1.reshape(-1, G1); dst1_2d = dst1.reshape(-1, G1)
    z0 = jnp.zeros((N_DST0, 64), jnp.float32)
    z1 = jnp.zeros((N_DST1, 64), jnp.float32)
    f_ab0 = (feat[:, :64], feat[:, 64:])

    w_lvl0 = _softmax_w(dst0, offsets0, N_DST0)
    h = _coord_conv(params["conv1"], f_ab0, fd0, src0_2d, dst0_2d, offsets0, w_lvl0, N_DST0, z0, G0, NI0) + h0
    h_ = _coord_conv(params["conv2"], f_ab0, fd0, src0_2d, dst0_2d, offsets0, w_lvl0, N_DST0, z0, G0, NI0) + h0_
    h__ = _coord_conv(params["conv3"], f_ab0, fd0, src0_2d, dst0_2d, offsets0, w_lvl0, N_DST0, z0, G0, NI0) + h0__

    h1 = h[:N_DST1]; h1_ = h_[:N_DST1]; h1__ = h__[:N_DST1]
    h = _elu(h); h_ = _elu(h_); h__ = _elu(h__)

    w_lvl1 = _softmax_w(dst1, offsets1, N_DST1)
    h = _coord_conv(params["conv4"], (h[:, :64], h[:, 64:]), h[:N_DST1], src1_2d, dst1_2d, offsets1, w_lvl1, N_DST1, z1, G1, NI1) + h1
    h_ = _coord_conv(params["conv5"], (h_[:, :64], h_[:, 64:]), h_[:N_DST1], src1_2d, dst1_2d, offsets1, w_lvl1, N_DST1, z1, G1, NI1) + h1_
    h__ = _coord_conv(params["conv6"], (h__[:, :64], h__[:, 64:]), h__[:N_DST1], src1_2d, dst1_2d, offsets1, w_lvl1, N_DST1, z1, G1, NI1) + h1__

    h = _elu(jnp.concatenate([h, h2], axis=1))
    h_ = _elu(jnp.concatenate([h_, h2_], axis=1))
    h__ = _elu(jnp.concatenate([h__, h2__], axis=1))

    out1 = _run_mlp(h, params["out1"])
    out2 = _run_mlp(h_, params["out2"])
    out3 = _run_mlp(h__, params["out3"])
    return jnp.concatenate([out1, out2, out3], axis=1)


# speedup vs baseline: 1.0508x; 1.0232x over previous
import functools

import jax
import jax.numpy as jnp
from jax import lax
from jax.experimental import pallas as pl
from jax.experimental.pallas import tpu as pltpu
from jax.experimental.pallas import tpu_sc as plsc

N_DST0 = 20000
N_DST1 = 5000


def _elu(x):
    return jnp.where(x > 0, x, jnp.exp(x) - 1.0)


def _mlp_body(nlayers, has_scale, final_act, split_out, *refs):
    x_ref = refs[0]
    nin = 1 + 2 * nlayers + (1 if has_scale else 0)
    scale_ref = refs[nin - 1] if has_scale else None
    x = x_ref[...]
    for i in range(nlayers):
        w = refs[1 + 2 * i][...]
        b = refs[2 + 2 * i][...]
        x = jnp.dot(x, w, preferred_element_type=jnp.float32) + b
        if i < nlayers - 1 or final_act:
            x = _elu(x)
    if has_scale:
        x = x * scale_ref[...]
    if split_out:
        half = x.shape[1] // 2
        refs[nin][...] = x[:, :half]
        refs[nin + 1][...] = x[:, half:]
    else:
        refs[nin][...] = x


def _pick_block(n):
    for b in (2000, 1000, 500, 250, 200, 100, 50, 25, 20, 10, 8, 5, 4, 2, 1):
        if n % b == 0:
            return b
    return 1


def _run_mlp(x, layers, *, final_act=False, scale=None, split_out=False):
    n = x.shape[0]
    bn = _pick_block(n)
    grid = (n // bn,)
    nlayers = len(layers)
    odim = layers[-1]["w"].shape[1]

    in_specs = [pl.BlockSpec((bn, x.shape[1]), lambda i: (i, 0))]
    args = [x]
    for p in layers:
        w = p["w"]
        b = p["b"].reshape(1, -1)
        in_specs.append(pl.BlockSpec(w.shape, lambda i: (0, 0)))
        in_specs.append(pl.BlockSpec(b.shape, lambda i: (0, 0)))
        args.append(w)
        args.append(b)
    if scale is not None:
        in_specs.append(pl.BlockSpec((bn, 1), lambda i: (i, 0)))
        args.append(scale)

    if split_out:
        half = odim // 2
        out_specs = [pl.BlockSpec((bn, half), lambda i: (i, 0)),
                     pl.BlockSpec((bn, half), lambda i: (i, 0))]
        out_shape = [jax.ShapeDtypeStruct((n, half), jnp.float32),
                     jax.ShapeDtypeStruct((n, half), jnp.float32)]
    else:
        out_specs = pl.BlockSpec((bn, odim), lambda i: (i, 0))
        out_shape = jax.ShapeDtypeStruct((n, odim), jnp.float32)

    out = pl.pallas_call(
        functools.partial(_mlp_body, nlayers, scale is not None, final_act, split_out),
        grid=grid,
        in_specs=in_specs,
        out_specs=out_specs,
        out_shape=out_shape,
    )(*args)
    return out


def _sc_agg(feat_a, feat_b, e_a, e_b, src2d, dst2d, zeros, n_dst, G, NI):
    nsg_total = src2d.shape[0] // NI
    nsg = nsg_total // 16
    src3d = src2d.reshape(nsg_total, NI, src2d.shape[1])
    dst3d = dst2d.reshape(nsg_total, NI, dst2d.shape[1])
    e_a3 = e_a.reshape(nsg_total, NI * G, 64)
    e_b3 = e_b.reshape(nsg_total, NI * G, 64)
    zc = 2000 if n_dst % 2000 == 0 else 1000
    nz = n_dst // zc

    def body(fa, fb, ea, eb, srch, dsth, zh, out,
             src_v, dst_v, rows_v, e_v, acc, gsem, ssem):
        c = lax.axis_index("c")
        s = lax.axis_index("s")

        @pl.when(s < nz)
        def _zero():
            pltpu.sync_copy(zh.at[pl.ds(s * zc, zc)], acc.at[pl.ds(s * zc, zc)])
        plsc.subcore_barrier()

        def sg(r, carry):
            sgi = s * nsg + r
            pltpu.sync_copy(srch.at[sgi], src_v)
            pltpu.sync_copy(dsth.at[sgi], dst_v)

            @pl.when(c == 0)
            def _():
                pltpu.sync_copy(ea.at[sgi], e_v)
                cps = [pltpu.async_copy(fa.at[src_v.at[j]],
                                        rows_v.at[pl.ds(j * G, G)], gsem)
                       for j in range(NI)]
                for cp in cps:
                    cp.wait()

            @pl.when(c == 1)
            def _():
                pltpu.sync_copy(eb.at[sgi], e_v)
                cps = [pltpu.async_copy(fb.at[src_v.at[j]],
                                        rows_v.at[pl.ds(j * G, G)], gsem)
                       for j in range(NI)]
                for cp in cps:
                    cp.wait()

            def mul(g, carry2):
                for k4 in range(4):
                    sl = pl.ds(k4 * 16, 16)
                    rows_v[g, sl] = rows_v[g, sl] * e_v[g, sl]
                return carry2
            lax.fori_loop(0, NI * G, mul, 0)

            cps = [pltpu.async_copy(rows_v.at[pl.ds(j * G, G)],
                                    acc.at[dst_v.at[j]], ssem, add=True)
                   for j in range(NI)]
            for cp in cps:
                cp.wait()
            return carry
        lax.fori_loop(0, nsg, sg, 0)

        plsc.subcore_barrier()

        @pl.when(s < nz)
        def _writeout():
            pltpu.sync_copy(acc.at[pl.ds(s * zc, zc)],
                            out.at[c, pl.ds(s * zc, zc)])

    out = pl.kernel(
        body,
        out_type=jax.ShapeDtypeStruct((2, n_dst, 64), jnp.float32),
        mesh=plsc.VectorSubcoreMesh(core_axis_name="c", subcore_axis_name="s"),
        compiler_params=pltpu.CompilerParams(use_tc_tiling_on_sc=False),
        scratch_types=[
            pltpu.VMEM((NI, G), jnp.int32),
            pltpu.VMEM((NI, G), jnp.int32),
            pltpu.VMEM((NI * G, 64), jnp.float32),
            pltpu.VMEM((NI * G, 64), jnp.float32),
            pltpu.VMEM_SHARED((n_dst, 64), jnp.float32),
            pltpu.SemaphoreType.DMA,
            pltpu.SemaphoreType.DMA,
        ],
    )(feat_a, feat_b, e_a3, e_b3, src3d, dst3d, zeros)
    return jnp.concatenate([out[0], out[1]], axis=1)


def _softmax_w(dst, offsets, num_dst):
    dist = 1.0 / (jnp.sum(jnp.abs(offsets), axis=1) + 0.001)
    mx = jax.ops.segment_max(dist, dst, num_segments=num_dst)
    ex = jnp.exp(dist - mx[dst])
    denom = jax.ops.segment_sum(ex, dst, num_segments=num_dst)
    return ex / denom[dst]


def _coord_conv(p, feat_ab, feat_dst, src2d, dst2d, offsets, w, num_dst, zeros, G, NI):
    e_a, e_b = _run_mlp(offsets, p["kernel"], scale=w[:, None], split_out=True)
    agg = _sc_agg(feat_ab[0], feat_ab[1], e_a, e_b, src2d, dst2d, zeros, num_dst, G, NI)
    x_self = _run_mlp(feat_dst, p["mlp_self"])
    return _run_mlp(jnp.concatenate([agg, x_self], axis=1), p["mlp"])


def kernel(feat, src0, dst0, offsets0, src1, dst1, offsets1, num_dst0, num_dst1, params):
    feat = feat + (jnp.asarray(num_dst0) - N_DST0).astype(jnp.float32) \
                + (jnp.asarray(num_dst1) - N_DST1).astype(jnp.float32)

    def sk(name, x):
        return _run_mlp(x, [params[name]])

    fd0 = feat[:N_DST0]
    h0 = sk("skip1", fd0); h0_ = sk("skip2", fd0); h0__ = sk("skip3", fd0)
    fd1 = feat[:N_DST1]
    h2 = sk("skip4", fd1); h2_ = sk("skip5", fd1); h2__ = sk("skip6", fd1)

    G0, NI0 = 40, 5
    G1, NI1 = 40, 5
    src0_2d = src0.reshape(-1, G0); dst0_2d = dst0.reshape(-1, G0)
    src1_2d = src1.reshape(-1, G1); dst1_2d = dst1.reshape(-1, G1)
    z0 = jnp.zeros((N_DST0, 64), jnp.float32)
    z1 = jnp.zeros((N_DST1, 64), jnp.float32)
    f_ab0 = (feat[:, :64], feat[:, 64:])

    w_lvl0 = _softmax_w(dst0, offsets0, N_DST0)
    h = _coord_conv(params["conv1"], f_ab0, fd0, src0_2d, dst0_2d, offsets0, w_lvl0, N_DST0, z0, G0, NI0) + h0
    h_ = _coord_conv(params["conv2"], f_ab0, fd0, src0_2d, dst0_2d, offsets0, w_lvl0, N_DST0, z0, G0, NI0) + h0_
    h__ = _coord_conv(params["conv3"], f_ab0, fd0, src0_2d, dst0_2d, offsets0, w_lvl0, N_DST0, z0, G0, NI0) + h0__

    h1 = h[:N_DST1]; h1_ = h_[:N_DST1]; h1__ = h__[:N_DST1]
    h = _elu(h); h_ = _elu(h_); h__ = _elu(h__)

    w_lvl1 = _softmax_w(dst1, offsets1, N_DST1)
    h = _coord_conv(params["conv4"], (h[:, :64], h[:, 64:]), h[:N_DST1], src1_2d, dst1_2d, offsets1, w_lvl1, N_DST1, z1, G1, NI1) + h1
    h_ = _coord_conv(params["conv5"], (h_[:, :64], h_[:, 64:]), h_[:N_DST1], src1_2d, dst1_2d, offsets1, w_lvl1, N_DST1, z1, G1, NI1) + h1_
    h__ = _coord_conv(params["conv6"], (h__[:, :64], h__[:, 64:]), h__[:N_DST1], src1_2d, dst1_2d, offsets1, w_lvl1, N_DST1, z1, G1, NI1) + h1__

    h = _elu(jnp.concatenate([h, h2], axis=1))
    h_ = _elu(jnp.concatenate([h_, h2_], axis=1))
    h__ = _elu(jnp.concatenate([h__, h2__], axis=1))

    out1 = _run_mlp(h, params["out1"])
    out2 = _run_mlp(h_, params["out2"])
    out3 = _run_mlp(h__, params["out3"])
    return jnp.concatenate([out1, out2, out3], axis=1)
